# Initial kernel scaffold; baseline (speedup 1.0000x reference)
import jax
import jax.numpy as jnp
from jax.experimental import pallas as pl


def kernel(x, edge_index, edge_weight, W1, b1, W2, b2):
    raise NotImplementedError("write your pallas kernel here")


# speedup vs baseline: 9.9416x; 9.9416x over previous
import functools
import jax
import jax.numpy as jnp
from jax import lax
from jax.experimental import pallas as pl
from jax.experimental.pallas import tpu as pltpu
from jax.experimental.pallas import tpu_sc as plsc

N = 10000
E = 320000
DP = 128
NC, NS = 2, 16
NW = NC * NS
EPW = E // NW
CHUNK = 80
NCHUNK = EPW // CHUNK
NPAD = 10240
RPT = NPAD // NS

_mesh = plsc.VectorSubcoreMesh(core_axis_name="c", subcore_axis_name="s")


@functools.partial(
    pl.kernel,
    out_type=jax.ShapeDtypeStruct((NC, NPAD), jnp.float32),
    mesh=_mesh,
    scratch_types=[
        pltpu.VMEM((CHUNK,), jnp.int32),
        pltpu.VMEM((CHUNK,), jnp.float32),
        pltpu.VMEM((RPT,), jnp.float32),
        pltpu.VMEM_SHARED((NPAD,), jnp.float32),
    ],
)
def _sc_degree(dst_hbm, w_hbm, out_hbm, idx_v, w_v, zbuf, acc):
    cid = lax.axis_index("c")
    sid = lax.axis_index("s")
    wid = cid * NS + sid

    zero16 = jnp.zeros((16,), jnp.float32)
    for k in range(RPT // 16):
        zbuf[pl.ds(k * 16, 16)] = zero16
    pltpu.sync_copy(zbuf, acc.at[pl.ds(sid * RPT, RPT)])
    plsc.subcore_barrier()

    def body(i, _):
        base = wid * EPW + i * CHUNK
        pltpu.sync_copy(dst_hbm.at[pl.ds(base, CHUNK)], idx_v)
        pltpu.sync_copy(w_hbm.at[pl.ds(base, CHUNK)], w_v)
        pltpu.sync_copy(w_v, acc.at[idx_v], add=True)
        return _

    lax.fori_loop(0, NCHUNK, body, None)
    plsc.subcore_barrier()
    pltpu.sync_copy(acc.at[pl.ds(sid * RPT, RPT)],
                    out_hbm.at[cid, pl.ds(sid * RPT, RPT)])


@functools.partial(
    pl.kernel,
    out_type=jax.ShapeDtypeStruct((NC, NPAD, DP), jnp.float32),
    mesh=_mesh,
    scratch_types=[
        pltpu.VMEM((CHUNK,), jnp.int32),
        pltpu.VMEM((CHUNK,), jnp.int32),
        pltpu.VMEM((CHUNK,), jnp.float32),
        pltpu.VMEM((CHUNK, DP), jnp.float32),
        pltpu.VMEM((16, DP), jnp.float32),
        pltpu.VMEM_SHARED((NPAD, DP), jnp.float32),
        pltpu.SemaphoreType.DMA,
    ],
)
def _sc_aggregate(y_hbm, src_hbm, dst_hbm, w_hbm, out_hbm,
                  sidx_v, didx_v, w_v, rows_v, zbuf, acc, sem):
    cid = lax.axis_index("c")
    sid = lax.axis_index("s")
    wid = cid * NS + sid

    zero16 = jnp.zeros((16,), jnp.float32)
    for r in range(16):
        for j in range(DP // 16):
            zbuf[r, pl.ds(j * 16, 16)] = zero16
    for k in range(RPT // 16):
        pltpu.sync_copy(zbuf, acc.at[pl.ds(sid * RPT + k * 16, 16)])
    plsc.subcore_barrier()

    def body(i, _):
        base = wid * EPW + i * CHUNK
        pltpu.sync_copy(src_hbm.at[pl.ds(base, CHUNK)], sidx_v)
        pltpu.sync_copy(dst_hbm.at[pl.ds(base, CHUNK)], didx_v)
        pltpu.sync_copy(w_hbm.at[pl.ds(base, CHUNK)], w_v)
        pltpu.async_copy(y_hbm.at[sidx_v], rows_v, sem).wait()
        for g in range(CHUNK // 16):
            wv = w_v[pl.ds(g * 16, 16)]
            for i in range(16):
                e = g * 16 + i
                bw = lax.gather(
                    wv, jnp.full((16, 1), i, jnp.int32),
                    lax.GatherDimensionNumbers(
                        offset_dims=(), collapsed_slice_dims=(0,),
                        start_index_map=(0,)),
                    (1,), mode=lax.GatherScatterMode.PROMISE_IN_BOUNDS)
                for j in range(DP // 16):
                    sl = pl.ds(j * 16, 16)
                    rows_v[e, sl] = rows_v[e, sl] * bw
        pltpu.sync_copy(rows_v, acc.at[didx_v], add=True)
        return _

    lax.fori_loop(0, NCHUNK, body, None)
    plsc.subcore_barrier()
    pltpu.sync_copy(acc.at[pl.ds(sid * RPT, RPT)],
                    out_hbm.at[cid, pl.ds(sid * RPT, RPT)])


R = 1000


def _tc1_body(dp_ref, x_ref, w1_ref, y1_ref):
    deg = dp_ref[:, 0] + dp_ref[:, 1] + 1.0
    dinv = jnp.where(deg > 0, lax.rsqrt(deg), 0.0)
    xw = jnp.dot(x_ref[...], w1_ref[...], preferred_element_type=jnp.float32)
    y1_ref[...] = dinv[:, None] * xw


def _tc2_body(dp_ref, p_ref, y1_ref, b1_ref, w2_ref, y2_ref):
    deg = dp_ref[:, 0] + dp_ref[:, 1] + 1.0
    dinv = jnp.where(deg > 0, lax.rsqrt(deg), 0.0)
    agg = p_ref[0] + p_ref[1] + y1_ref[...]
    h = dinv[:, None] * agg + b1_ref[...]
    h = jnp.where(h >= 0, h, 0.01 * h)
    hw = jnp.dot(h, w2_ref[...], preferred_element_type=jnp.float32)
    y2_ref[...] = dinv[:, None] * hw


def _tc3_body(dp_ref, p_ref, y2_ref, b2_ref, out_ref):
    deg = dp_ref[:, 0] + dp_ref[:, 1] + 1.0
    dinv = jnp.where(deg > 0, lax.rsqrt(deg), 0.0)
    agg = p_ref[0] + p_ref[1] + y2_ref[...]
    out_ref[...] = dinv[:, None] * agg + b2_ref[...]


def _row_spec(width):
    return pl.BlockSpec((R, width), lambda i: (i, 0))


_dp_spec = pl.BlockSpec((R, 2), lambda i: (i, 0))
_p_spec = pl.BlockSpec((2, R, DP), lambda i: (0, i, 0))
_w_spec = pl.BlockSpec((DP, DP), lambda i: (0, 0))
_b_spec = pl.BlockSpec((1, DP), lambda i: (0, 0))

_tc1 = pl.pallas_call(
    _tc1_body,
    grid=(N // R,),
    in_specs=[_dp_spec, _row_spec(DP), _w_spec],
    out_specs=_row_spec(DP),
    out_shape=jax.ShapeDtypeStruct((N, DP), jnp.float32),
)

_tc2 = pl.pallas_call(
    _tc2_body,
    grid=(N // R,),
    in_specs=[_dp_spec, _p_spec, _row_spec(DP), _b_spec, _w_spec],
    out_specs=_row_spec(DP),
    out_shape=jax.ShapeDtypeStruct((N, DP), jnp.float32),
)

_tc3 = pl.pallas_call(
    _tc3_body,
    grid=(N // R,),
    in_specs=[_dp_spec, _p_spec, _row_spec(DP), _b_spec],
    out_specs=_row_spec(DP),
    out_shape=jax.ShapeDtypeStruct((N, DP), jnp.float32),
)


def kernel(x, edge_index, edge_weight, W1, b1, W2, b2):
    src = edge_index[0].astype(jnp.int32)
    dst = edge_index[1].astype(jnp.int32)
    w = edge_weight.astype(jnp.float32)

    hidden = W1.shape[1]
    w1p = jnp.zeros((DP, DP), jnp.float32).at[:, :hidden].set(W1)
    b1p = jnp.zeros((1, DP), jnp.float32).at[0, :hidden].set(b1)
    w2p = jnp.zeros((DP, DP), jnp.float32).at[:hidden, :].set(W2)
    b2p = b2.reshape(1, DP)

    deg_p = _sc_degree(dst, w)[:, :N].T
    y1 = _tc1(deg_p, x, w1p)
    p1 = _sc_aggregate(y1, src, dst, w)[:, :N]
    y2 = _tc2(deg_p, p1, y1, b1p, w2p)
    p2 = _sc_aggregate(y2, src, dst, w)[:, :N]
    out = _tc3(deg_p, p2, y2, b2p)
    return out
